# baseline (speedup 1.0000x reference)
import jax
import jax.numpy as jnp
from jax import lax
from jax.experimental import pallas as pl
from jax.experimental.pallas import tpu as pltpu
from jax.experimental.pallas import tpu_sc as plsc

_N = 10000
_K = 27
_HID = 128
_CIN = 256
_E = 160000
_NK = _N * _K
_KH = _K * _HID

_BLK = 128
_NSUB = 16
_EP = 163840
_NBLK = _EP // _BLK
_SB = 4
_BN = 400
_BD = 1152

_mesh = plsc.VectorSubcoreMesh(core_axis_name="c", subcore_axis_name="s")

_NBT = _NBLK // 2 // _NSUB
_NSB = _NBT // _SB


def _sc_body(edg_hbm, tab_hbm, out_hbm,
             edg0, edg1, idx0, idx1, rows0, rows1,
             accum, isem0, isem1, gsem0, gsem1):
    c = lax.axis_index("c")
    s = lax.axis_index("s")

    blk0 = c * (_NBLK // 2) + s * _NBT

    pltpu.async_copy(edg_hbm.at[pl.ds(blk0, _SB)], edg0, isem0)

    def _zrow(i, _):
        r = i // 8
        col = (i % 8) * 16
        rows0[r, pl.ds(col, 16)] = jnp.zeros((16,), jnp.float32)
        return 0
    lax.fori_loop(0, _BLK * 8, _zrow, 0)

    @pl.when(s < 10)
    def _init():
        base = s * 1000
        for j in range(7):
            pltpu.sync_copy(rows0, accum.at[pl.ds(base + j * 128, 128)])
        pltpu.sync_copy(rows0.at[pl.ds(0, 104)],
                        accum.at[pl.ds(base + 896, 104)])

    plsc.subcore_barrier()

    rows = (rows0, rows1)
    gsems = (gsem0, gsem1)

    def _superblock(q, eb, isem, oeb, oisem, ib):
        pltpu.make_async_copy(edg_hbm.at[pl.ds(blk0 + q * _SB, _SB)],
                              eb, isem).wait()
        @pl.when(q < _NSB - 1)
        def _():
            pltpu.async_copy(edg_hbm.at[pl.ds(blk0 + (q + 1) * _SB, _SB)],
                             oeb, oisem)
        def _mkidx(b, _):
            for j in range(_BLK // 16):
                sl = pl.ds(j * 16, 16)
                ib[b, sl] = eb[b, 0, sl] * _K + eb[b, 1, sl]
            return 0
        lax.fori_loop(0, _SB, _mkidx, 0)
        pltpu.async_copy(tab_hbm.at[ib.at[0]], rows[0], gsems[0])
        for k in range(_SB):
            if k + 1 < _SB:
                pltpu.async_copy(tab_hbm.at[ib.at[k + 1]],
                                 rows[(k + 1) % 2], gsems[(k + 1) % 2])
            pltpu.make_async_copy(tab_hbm.at[ib.at[k]],
                                  rows[k % 2], gsems[k % 2]).wait()
            pltpu.sync_copy(rows[k % 2], accum.at[eb.at[k, 2]], add=True)

    def _pair(p, _):
        _superblock(2 * p, edg0, isem0, edg1, isem1, idx0)
        _superblock(2 * p + 1, edg1, isem1, edg0, isem0, idx1)
        return 0
    lax.fori_loop(0, _NSB // 2, _pair, 0)

    plsc.subcore_barrier()

    @pl.when(s < 10)
    def _flush():
        base = s * 1000
        pltpu.sync_copy(accum.at[pl.ds(base, 1000)],
                        out_hbm.at[pl.ds(c * _N + base, 1000)])


_sc_pass = pl.kernel(
    _sc_body,
    out_type=jax.ShapeDtypeStruct((2 * _N, _HID), jnp.float32),
    mesh=_mesh,
    scratch_types=[
        pltpu.VMEM((_SB, 3, _BLK), jnp.int32),
        pltpu.VMEM((_SB, 3, _BLK), jnp.int32),
        pltpu.VMEM((_SB, _BLK), jnp.int32),
        pltpu.VMEM((_SB, _BLK), jnp.int32),
        pltpu.VMEM((_BLK, _HID), jnp.float32),
        pltpu.VMEM((_BLK, _HID), jnp.float32),
        pltpu.VMEM_SHARED((_N + 8, _HID), jnp.float32),
        pltpu.SemaphoreType.DMA,
        pltpu.SemaphoreType.DMA,
        pltpu.SemaphoreType.DMA,
        pltpu.SemaphoreType.DMA,
    ],
)


def _mm_body(lhs_ref, w_ref, out_ref):
    out_ref[...] = jnp.dot(lhs_ref[...], w_ref[...],
                           preferred_element_type=jnp.float32)


_mm = pl.pallas_call(
    _mm_body,
    grid=(_KH // _BD, _N // _BN),
    in_specs=[
        pl.BlockSpec((_BN, _CIN), lambda j, i: (i, 0)),
        pl.BlockSpec((_CIN, _BD), lambda j, i: (0, j)),
    ],
    out_specs=pl.BlockSpec((_BN, _BD), lambda j, i: (i, j)),
    out_shape=jax.ShapeDtypeStruct((_N, _KH), jnp.float32),
)


def _q_mm_body(p0_ref, p1_ref, h_ref, x_ref, br_ref, wh_ref, wx_ref, out_ref):
    r = jax.nn.sigmoid(p0_ref[...] + p1_ref[...] + br_ref[0])
    rh = (r * h_ref[...]).astype(jnp.bfloat16)
    xb = x_ref[...].astype(jnp.bfloat16)
    out_ref[...] = (
        jnp.dot(rh, wh_ref[...], preferred_element_type=jnp.float32)
        + jnp.dot(xb, wx_ref[...], preferred_element_type=jnp.float32))


_q_mm = pl.pallas_call(
    _q_mm_body,
    grid=(_KH // _BD, _N // _BN),
    in_specs=[
        pl.BlockSpec((_BN, _HID), lambda j, i: (i, 0)),
        pl.BlockSpec((_BN, _HID), lambda j, i: (i, 0)),
        pl.BlockSpec((_BN, _HID), lambda j, i: (i, 0)),
        pl.BlockSpec((_BN, _HID), lambda j, i: (i, 0)),
        pl.BlockSpec((1, _HID), lambda j, i: (0, 0)),
        pl.BlockSpec((_HID, _BD), lambda j, i: (0, j)),
        pl.BlockSpec((_HID, _BD), lambda j, i: (0, j)),
    ],
    out_specs=pl.BlockSpec((_BN, _BD), lambda j, i: (i, j)),
    out_shape=jax.ShapeDtypeStruct((_N, _KH), jnp.float32),
)


def _gate_body(z0_ref, z1_ref, q0_ref, q1_ref, h_ref, bz_ref, bq_ref, out_ref):
    z = jax.nn.sigmoid(z0_ref[...] + z1_ref[...] + bz_ref[0])
    q = jnp.tanh(q0_ref[...] + q1_ref[...] + bq_ref[0])
    out_ref[...] = (1.0 - z) * h_ref[...] + z * q


_gate = pl.pallas_call(
    _gate_body,
    grid=(_N // _BN,),
    in_specs=[
        pl.BlockSpec((_BN, _HID), lambda i: (i, 0)),
        pl.BlockSpec((_BN, _HID), lambda i: (i, 0)),
        pl.BlockSpec((_BN, _HID), lambda i: (i, 0)),
        pl.BlockSpec((_BN, _HID), lambda i: (i, 0)),
        pl.BlockSpec((_BN, _HID), lambda i: (i, 0)),
        pl.BlockSpec((1, _HID), lambda i: (0, 0)),
        pl.BlockSpec((1, _HID), lambda i: (0, 0)),
    ],
    out_specs=pl.BlockSpec((_BN, _HID), lambda i: (i, 0)),
    out_shape=jax.ShapeDtypeStruct((_N, _HID), jnp.float32),
)


def kernel(h, x, edge_index, edge_kernel, Wz, bz, Wr, br, Wq, bq):
    hxb = jnp.concatenate([h, x], axis=1).astype(jnp.bfloat16)
    wzf = Wz.transpose(1, 0, 2).reshape(_CIN, _KH).astype(jnp.bfloat16)
    wrf = Wr.transpose(1, 0, 2).reshape(_CIN, _KH).astype(jnp.bfloat16)
    wqh = Wq[:, :_HID, :].transpose(1, 0, 2).reshape(_HID, _KH).astype(
        jnp.bfloat16)
    wqx = Wq[:, _HID:, :].transpose(1, 0, 2).reshape(_HID, _KH).astype(
        jnp.bfloat16)

    nchunk = 32
    chunk = _E // nchunk
    cpad = (_EP - _E) // nchunk
    src = jnp.pad(edge_index[0].reshape(nchunk, chunk), ((0, 0), (0, cpad)))
    kern = jnp.pad(edge_kernel.reshape(nchunk, chunk), ((0, 0), (0, cpad)))
    dpad = jnp.broadcast_to(_N + (jnp.arange(cpad, dtype=jnp.int32) % 8),
                            (nchunk, cpad))
    dst = jnp.concatenate([edge_index[1].reshape(nchunk, chunk), dpad], axis=1)
    edg = jnp.stack([src.reshape(_NBLK, _BLK), kern.reshape(_NBLK, _BLK),
                     dst.reshape(_NBLK, _BLK)], axis=1)

    t1r = _mm(hxb, wrf).reshape(_NK, _HID)
    pr = _sc_pass(edg, t1r)
    t1z = _mm(hxb, wzf).reshape(_NK, _HID)
    pz = _sc_pass(edg, t1z)
    t2 = _q_mm(pr[:_N], pr[_N:], h, x, br.reshape(1, _HID),
               wqh, wqx).reshape(_NK, _HID)
    qp = _sc_pass(edg, t2)
    return _gate(pz[:_N], pz[_N:], qp[:_N], qp[_N:], h,
                 bz.reshape(1, _HID), bq.reshape(1, _HID))
